# Initial kernel scaffold; baseline (speedup 1.0000x reference)
import functools
import math

import jax
import jax.numpy as jnp
from jax import lax
from jax.experimental import pallas as pl
from jax.experimental.pallas import tpu as pltpu
from jax.experimental.pallas import tpu_sc as plsc

N = 10000
E = 320000
HC = 128
NRBF = 64
D3 = 3 * HC

NC = 2
NS = 16
NW = NC * NS
EP_T = E // NW
EP_S = E // NS
NHALF = N // NC
ACC_ROWS = 5120
TRASH = NHALF
GCH = 200
SCH = 160

_INV3 = 1.0 / math.sqrt(3.0)
_INVH = 1.0 / math.sqrt(HC)


def _gather_bin_kernel(x_hbm, vec_hbm, j_hbm, i_hbm,
                       gx_hbm, gv_hbm, bid_hbm, bdst_hbm, bcnt_hbm):
    c = lax.axis_index("c")
    s = lax.axis_index("s")
    wid = c * NS + s

    def bin_phase(ivm, idsb, dstb, cbuf):
        pltpu.sync_copy(i_hbm.at[pl.ds(s * EP_S, EP_S)], ivm)

        def memset(k, _):
            idsb[pl.ds(k * 16, 16)] = jnp.zeros((16,), jnp.int32)
            dstb[pl.ds(k * 16, 16)] = jnp.full((16,), TRASH, jnp.int32)
            return 0

        lax.fori_loop(0, (EP_S + 16) // 16, memset, 0)

        lo = c * NHALF

        def body(k, cnt):
            iv = ivm[pl.ds(k * 16, 16)]
            m = (iv >= lo) & (iv < lo + NHALF)
            dl = iv - lo
            ids = s * EP_S + k * 16 + lax.iota(jnp.int32, 16)
            plsc.store_compressed(idsb.at[pl.ds(cnt, 16)], ids, m)
            plsc.store_compressed(dstb.at[pl.ds(cnt, 16)], dl, m)
            pc = jnp.sum(jnp.where(m, jnp.int32(1), jnp.int32(0)))
            return cnt + pc

        cnt = lax.fori_loop(0, EP_S // 16, body, jnp.int32(0))

        pltpu.sync_copy(idsb.at[pl.ds(0, EP_S)], bid_hbm.at[wid])
        pltpu.sync_copy(dstb.at[pl.ds(0, EP_S)], bdst_hbm.at[wid])
        cbuf[...] = jnp.where(lax.iota(jnp.int32, 16) == 0,
                              jnp.full((16,), cnt, jnp.int32),
                              jnp.zeros((16,), jnp.int32))
        pltpu.sync_copy(cbuf, bcnt_hbm.at[wid])

    pl.run_scoped(bin_phase,
                  pltpu.VMEM((EP_S,), jnp.int32),
                  pltpu.VMEM((EP_S + 16,), jnp.int32),
                  pltpu.VMEM((EP_S + 16,), jnp.int32),
                  pltpu.VMEM((16,), jnp.int32))

    def gather_phase(jvm, gx, gv, sem1, sem2):
        pltpu.sync_copy(j_hbm.at[pl.ds(wid * EP_T, EP_T)], jvm)

        def body(t, _):
            base = wid * EP_T + t * GCH
            idxs = jvm.at[pl.ds(t * GCH, GCH)]
            cp1 = pltpu.async_copy(x_hbm.at[idxs], gx, sem1)
            cp2 = pltpu.async_copy(vec_hbm.at[idxs], gv, sem2)
            cp1.wait()
            cp2.wait()
            pltpu.sync_copy(gx, gx_hbm.at[pl.ds(base, GCH)])
            pltpu.sync_copy(gv, gv_hbm.at[pl.ds(base, GCH)])
            return 0

        lax.fori_loop(0, EP_T // GCH, body, 0)

    pl.run_scoped(gather_phase,
                  pltpu.VMEM((EP_T,), jnp.int32),
                  pltpu.VMEM((GCH, HC), jnp.float32),
                  pltpu.VMEM((GCH, D3), jnp.float32),
                  pltpu.SemaphoreType.DMA,
                  pltpu.SemaphoreType.DMA)


def _edge_mlp_kernel(gx_ref, gv_ref, rbf_ref, ev0_ref, ev1_ref, ev2_ref,
                     W1_ref, b1_ref, W2_ref, b2_ref, Wr_ref, br_ref,
                     mv_ref, mx_ref):
    dn = (((1,), (1,)), ((), ()))
    gx = gx_ref[...]
    h = lax.dot_general(gx, W1_ref[...], dn, preferred_element_type=jnp.float32)
    h = h + b1_ref[...]
    h = (h * jax.nn.sigmoid(h)) * (1.0 / 0.6)
    xh = lax.dot_general(h, W2_ref[...], dn, preferred_element_type=jnp.float32)
    xh = xh + b2_ref[...]
    rh = lax.dot_general(rbf_ref[...], Wr_ref[...], dn,
                         preferred_element_type=jnp.float32)
    rh = rh + br_ref[...]
    t = xh * rh * _INV3
    t1 = t[:, :HC]
    t2 = t[:, HC:2 * HC]
    mx_ref[...] = t[:, 2 * HC:]
    evs = (ev0_ref[...], ev1_ref[...], ev2_ref[...])
    for d in range(3):
        mv_ref[:, d * HC:(d + 1) * HC] = (
            t1 * gv_ref[:, d * HC:(d + 1) * HC] + t2 * evs[d]) * _INVH


def _scatter_kernel(mv_hbm, mx_hbm, bid_hbm, bdst_hbm, bcnt_hbm,
                    dv_hbm, dx_hbm):
    c = lax.axis_index("c")
    s = lax.axis_index("s")
    wid = c * NS + s

    def outer(idsv, dstv, ichunk, dchunk, cbuf, sem):
        pltpu.sync_copy(bcnt_hbm.at[wid], cbuf)
        cnt = jnp.max(cbuf[...])
        nch = (cnt + (SCH - 1)) // SCH
        pltpu.sync_copy(bid_hbm.at[wid], idsv)
        pltpu.sync_copy(bdst_hbm.at[wid], dstv)

        def one_pass(src_hbm, out_hbm, width, acc, rows, zb):
            for r in range(16):
                for q in range(width // 16):
                    zb[r, pl.ds(q * 16, 16)] = jnp.zeros((16,), jnp.float32)
            rows_per_tile = ACC_ROWS // NS

            def zbody(t, _):
                pltpu.sync_copy(zb, acc.at[pl.ds(s * rows_per_tile + t * 16, 16)])
                return 0

            lax.fori_loop(0, rows_per_tile // 16, zbody, 0)
            plsc.subcore_barrier()

            def body(k, _):
                for q in range(SCH // 16):
                    ichunk[pl.ds(q * 16, 16)] = idsv[pl.ds(k * SCH + q * 16, 16)]
                    dchunk[pl.ds(q * 16, 16)] = dstv[pl.ds(k * SCH + q * 16, 16)]
                pltpu.async_copy(src_hbm.at[ichunk], rows, sem).wait()
                pltpu.sync_copy(rows, acc.at[dchunk], add=True)
                return 0

            lax.fori_loop(0, nch, body, 0)
            plsc.subcore_barrier()

            @pl.when(s < 8)
            def _():
                pltpu.sync_copy(acc.at[pl.ds(s * 625, 625)],
                                out_hbm.at[pl.ds(c * NHALF + s * 625, 625)])

            plsc.subcore_barrier()

        def pass_v(acc, rows, zb):
            one_pass(mv_hbm, dv_hbm, D3, acc, rows, zb)

        pl.run_scoped(pass_v,
                      plsc.MemoryRef((ACC_ROWS, D3), jnp.float32,
                                     pltpu.VMEM_SHARED),
                      pltpu.VMEM((SCH, D3), jnp.float32),
                      pltpu.VMEM((16, D3), jnp.float32))

        def pass_x(acc, rows, zb):
            one_pass(mx_hbm, dx_hbm, HC, acc, rows, zb)

        pl.run_scoped(pass_x,
                      plsc.MemoryRef((ACC_ROWS, HC), jnp.float32,
                                     pltpu.VMEM_SHARED),
                      pltpu.VMEM((SCH, HC), jnp.float32),
                      pltpu.VMEM((16, HC), jnp.float32))

    pl.run_scoped(outer,
                  pltpu.VMEM((EP_S,), jnp.int32),
                  pltpu.VMEM((EP_S,), jnp.int32),
                  pltpu.VMEM((SCH,), jnp.int32),
                  pltpu.VMEM((SCH,), jnp.int32),
                  pltpu.VMEM((16,), jnp.int32),
                  pltpu.SemaphoreType.DMA)


_SC_MESH = plsc.VectorSubcoreMesh(core_axis_name="c", subcore_axis_name="s")

_gather_call = functools.partial(
    pl.kernel, mesh=_SC_MESH,
    out_type=[
        jax.ShapeDtypeStruct((E, HC), jnp.float32),
        jax.ShapeDtypeStruct((E, D3), jnp.float32),
        jax.ShapeDtypeStruct((NW, EP_S), jnp.int32),
        jax.ShapeDtypeStruct((NW, EP_S), jnp.int32),
        jax.ShapeDtypeStruct((NW, 16), jnp.int32),
    ])(_gather_bin_kernel)

_scatter_call = functools.partial(
    pl.kernel, mesh=_SC_MESH,
    out_type=[
        jax.ShapeDtypeStruct((N, D3), jnp.float32),
        jax.ShapeDtypeStruct((N, HC), jnp.float32),
    ])(_scatter_kernel)

_EB = 2000


def _edge_mlp(gx, gv, rbf, ev0, ev1, ev2, W1, b1, W2, b2, Wr, br):
    grid = (E // _EB,)
    row = lambda m: (m, 0)
    fixed = lambda m: (0, 0)
    return pl.pallas_call(
        _edge_mlp_kernel,
        grid=grid,
        in_specs=[
            pl.BlockSpec((_EB, HC), row),
            pl.BlockSpec((_EB, D3), row),
            pl.BlockSpec((_EB, NRBF), row),
            pl.BlockSpec((_EB, 1), row),
            pl.BlockSpec((_EB, 1), row),
            pl.BlockSpec((_EB, 1), row),
            pl.BlockSpec((HC // 2, HC), fixed),
            pl.BlockSpec((1, HC // 2), fixed),
            pl.BlockSpec((D3, HC // 2), fixed),
            pl.BlockSpec((1, D3), fixed),
            pl.BlockSpec((D3, NRBF), fixed),
            pl.BlockSpec((1, D3), fixed),
        ],
        out_specs=[
            pl.BlockSpec((_EB, D3), row),
            pl.BlockSpec((_EB, HC), row),
        ],
        out_shape=[
            jax.ShapeDtypeStruct((E, D3), jnp.float32),
            jax.ShapeDtypeStruct((E, HC), jnp.float32),
        ],
    )(gx, gv, rbf, ev0, ev1, ev2, W1, b1, W2, b2, Wr, br)


def kernel(x, vec, edge_rbf, edge_vector, W1, b1, W2, b2, Wr, br, edge_index):
    vec2 = vec.reshape(N, D3)
    jj = edge_index[0]
    ii = edge_index[1]
    gx, gv, bid, bdst, bcnt = _gather_call(x, vec2, jj, ii)
    mv, mx = _edge_mlp(gx, gv, edge_rbf,
                       edge_vector[:, 0:1], edge_vector[:, 1:2],
                       edge_vector[:, 2:3],
                       W1, b1.reshape(1, -1), W2, b2.reshape(1, -1),
                       Wr, br.reshape(1, -1))
    dv2, dx = _scatter_call(mv, mx, bid, bdst, bcnt)
    return (dx, dv2.reshape(N, 3, HC))


# speedup vs baseline: 16.0775x; 16.0775x over previous
import functools
import math

import jax
import jax.numpy as jnp
from jax import lax
from jax.experimental import pallas as pl
from jax.experimental.pallas import tpu as pltpu
from jax.experimental.pallas import tpu_sc as plsc

N = 10000
E = 320000
HC = 128
NRBF = 64
D3 = 3 * HC

NC = 2
NS = 16
NW = NC * NS
EP_T = E // NW
EP_S = E // NS
NHALF = N // NC
ACC_ROWS = 5120
TRASH = NHALF
GCH = 80
SCH = 160
ICH = 2000

_PACK_SH = 13

_INV3 = 1.0 / math.sqrt(3.0)
_INVH = 1.0 / math.sqrt(HC)


def _gather_bin_kernel(x_hbm, vec_hbm, j_hbm, i_hbm,
                       gx_hbm, gv_hbm, bpk_hbm, bcnt_hbm):
    c = lax.axis_index("c")
    s = lax.axis_index("s")
    wid = c * NS + s

    def bin_phase(ivm, pb, cbuf):
        def memset(k, _):
            pb[pl.ds(k * 16, 16)] = jnp.full((16,), TRASH, jnp.int32)
            return 0

        lax.fori_loop(0, EP_S // 16, memset, 0)

        lo = c * NHALF

        def chunk(ci, cnt):
            pltpu.sync_copy(i_hbm.at[pl.ds(s * EP_S + ci * ICH, ICH)], ivm)

            def body(k, cnt):
                iv = ivm[pl.ds(k * 16, 16)]
                m = (iv >= lo) & (iv < lo + NHALF)
                dl = iv - lo
                ids = s * EP_S + ci * ICH + k * 16 + lax.iota(jnp.int32, 16)
                w = jnp.bitwise_or(jnp.left_shift(ids, _PACK_SH), dl)
                ps = plsc.cumsum(jnp.where(m, jnp.int32(1), jnp.int32(0)))
                pos = cnt + ps - 1
                plsc.store_scatter(pb, [pos], w, mask=m)
                return cnt + jnp.max(ps)

            return lax.fori_loop(0, ICH // 16, body, cnt)

        cnt = lax.fori_loop(0, EP_S // ICH, chunk, jnp.int32(0))

        pltpu.sync_copy(pb, bpk_hbm.at[wid])
        cbuf[...] = jnp.where(lax.iota(jnp.int32, 16) == 0,
                              jnp.full((16,), cnt, jnp.int32),
                              jnp.zeros((16,), jnp.int32))
        pltpu.sync_copy(cbuf, bcnt_hbm.at[wid])

    pl.run_scoped(bin_phase,
                  pltpu.VMEM((ICH,), jnp.int32),
                  pltpu.VMEM((EP_S,), jnp.int32),
                  pltpu.VMEM((16,), jnp.int32))

    def gather_phase(jvm, gx, gv, sem1, sem2):
        pltpu.sync_copy(j_hbm.at[pl.ds(wid * EP_T, EP_T)], jvm)

        def body(t, _):
            base = wid * EP_T + t * GCH
            idxs = jvm.at[pl.ds(t * GCH, GCH)]
            cp1 = pltpu.async_copy(x_hbm.at[idxs], gx, sem1)
            cp2 = pltpu.async_copy(vec_hbm.at[idxs], gv, sem2)
            cp1.wait()
            cp2.wait()
            pltpu.sync_copy(gx, gx_hbm.at[pl.ds(base, GCH)])
            pltpu.sync_copy(gv, gv_hbm.at[pl.ds(base, GCH)])
            return 0

        lax.fori_loop(0, EP_T // GCH, body, 0)

    pl.run_scoped(gather_phase,
                  pltpu.VMEM((EP_T,), jnp.int32),
                  pltpu.VMEM((GCH, HC), jnp.float32),
                  pltpu.VMEM((GCH, D3), jnp.float32),
                  pltpu.SemaphoreType.DMA,
                  pltpu.SemaphoreType.DMA)


def _edge_mlp_kernel(gx_ref, gv_ref, rbf_ref, ev0_ref, ev1_ref, ev2_ref,
                     W1_ref, b1_ref, W2_ref, b2_ref, Wr_ref, br_ref,
                     mv0_ref, mv1_ref, mv2_ref, mx_ref):
    dn = (((1,), (1,)), ((), ()))
    gx = gx_ref[...]
    h = lax.dot_general(gx, W1_ref[...], dn, preferred_element_type=jnp.float32)
    h = h + b1_ref[...]
    h = (h * jax.nn.sigmoid(h)) * (1.0 / 0.6)
    xh = lax.dot_general(h, W2_ref[...], dn, preferred_element_type=jnp.float32)
    xh = xh + b2_ref[...]
    rh = lax.dot_general(rbf_ref[...], Wr_ref[...], dn,
                         preferred_element_type=jnp.float32)
    rh = rh + br_ref[...]
    t = xh * rh * _INV3
    t1 = t[:, :HC]
    t2 = t[:, HC:2 * HC]
    mx_ref[...] = t[:, 2 * HC:]
    evs = (ev0_ref[...], ev1_ref[...], ev2_ref[...])
    mv_refs = (mv0_ref, mv1_ref, mv2_ref)
    for d in range(3):
        mv_refs[d][...] = (
            t1 * gv_ref[:, d * HC:(d + 1) * HC] + t2 * evs[d]) * _INVH


def _scatter_kernel(m0_hbm, m1_hbm, m2_hbm, m3_hbm, bpk_hbm, bcnt_hbm,
                    o0_hbm, o1_hbm, o2_hbm, o3_hbm, acc):
    c = lax.axis_index("c")
    s = lax.axis_index("s")
    wid = c * NS + s

    def outer(pbv, ichunk, dchunk, cbuf, rows, zb, sem):
        pltpu.sync_copy(bcnt_hbm.at[wid], cbuf)
        cnt = jnp.max(cbuf[...])
        nch = (cnt + (SCH - 1)) // SCH
        pltpu.sync_copy(bpk_hbm.at[wid], pbv)

        for r in range(16):
            for q in range(HC // 16):
                zb[r, pl.ds(q * 16, 16)] = jnp.zeros((16,), jnp.float32)

        rows_per_tile = ACC_ROWS // NS

        for src_hbm, out_hbm in ((m0_hbm, o0_hbm), (m1_hbm, o1_hbm),
                                 (m2_hbm, o2_hbm), (m3_hbm, o3_hbm)):
            def zbody(t, _):
                pltpu.sync_copy(zb,
                                acc.at[pl.ds(s * rows_per_tile + t * 16, 16)])
                return 0

            lax.fori_loop(0, rows_per_tile // 16, zbody, 0)
            plsc.subcore_barrier()

            def body(k, _):
                for q in range(SCH // 16):
                    w = pbv[pl.ds(k * SCH + q * 16, 16)]
                    ichunk[pl.ds(q * 16, 16)] = lax.shift_right_logical(
                        w, jnp.full((16,), _PACK_SH, jnp.int32))
                    dchunk[pl.ds(q * 16, 16)] = jnp.bitwise_and(
                        w, (1 << _PACK_SH) - 1)
                pltpu.async_copy(src_hbm.at[ichunk], rows, sem).wait()
                pltpu.sync_copy(rows, acc.at[dchunk], add=True)
                return 0

            lax.fori_loop(0, nch, body, 0)
            plsc.subcore_barrier()

            @pl.when(s < 15)
            def _():
                pltpu.sync_copy(acc.at[pl.ds(s * 312, 312)],
                                out_hbm.at[pl.ds(c * NHALF + s * 312, 312)])

            @pl.when(s == 15)
            def _():
                pltpu.sync_copy(acc.at[pl.ds(4680, 320)],
                                out_hbm.at[pl.ds(c * NHALF + 4680, 320)])

            plsc.subcore_barrier()

    pl.run_scoped(outer,
                  pltpu.VMEM((EP_S,), jnp.int32),
                  pltpu.VMEM((SCH,), jnp.int32),
                  pltpu.VMEM((SCH,), jnp.int32),
                  pltpu.VMEM((16,), jnp.int32),
                  pltpu.VMEM((SCH, HC), jnp.float32),
                  pltpu.VMEM((16, HC), jnp.float32),
                  pltpu.SemaphoreType.DMA)


_SC_MESH = plsc.VectorSubcoreMesh(core_axis_name="c", subcore_axis_name="s")
_SC_PARAMS = pltpu.CompilerParams(needs_layout_passes=False)

_gather_call = functools.partial(
    pl.kernel, mesh=_SC_MESH, compiler_params=_SC_PARAMS,
    out_type=[
        jax.ShapeDtypeStruct((E, HC), jnp.float32),
        jax.ShapeDtypeStruct((E, D3), jnp.float32),
        jax.ShapeDtypeStruct((NW, EP_S), jnp.int32),
        jax.ShapeDtypeStruct((NW, 16), jnp.int32),
    ])(_gather_bin_kernel)

_scatter_call = functools.partial(
    pl.kernel, mesh=_SC_MESH, compiler_params=_SC_PARAMS,
    out_type=[jax.ShapeDtypeStruct((N, HC), jnp.float32)] * 4,
    scratch_types=[pltpu.VMEM_SHARED((ACC_ROWS, HC), jnp.float32)],
    )(_scatter_kernel)

_EB = 2000


def _edge_mlp(gx, gv, rbf, ev0, ev1, ev2, W1, b1, W2, b2, Wr, br):
    grid = (E // _EB,)
    row = lambda m: (m, 0)
    fixed = lambda m: (0, 0)
    return pl.pallas_call(
        _edge_mlp_kernel,
        grid=grid,
        in_specs=[
            pl.BlockSpec((_EB, HC), row),
            pl.BlockSpec((_EB, D3), row),
            pl.BlockSpec((_EB, NRBF), row),
            pl.BlockSpec((_EB, 1), row),
            pl.BlockSpec((_EB, 1), row),
            pl.BlockSpec((_EB, 1), row),
            pl.BlockSpec((HC // 2, HC), fixed),
            pl.BlockSpec((1, HC // 2), fixed),
            pl.BlockSpec((D3, HC // 2), fixed),
            pl.BlockSpec((1, D3), fixed),
            pl.BlockSpec((D3, NRBF), fixed),
            pl.BlockSpec((1, D3), fixed),
        ],
        out_specs=[pl.BlockSpec((_EB, HC), row)] * 4,
        out_shape=[jax.ShapeDtypeStruct((E, HC), jnp.float32)] * 4,
    )(gx, gv, rbf, ev0, ev1, ev2, W1, b1, W2, b2, Wr, br)


def kernel(x, vec, edge_rbf, edge_vector, W1, b1, W2, b2, Wr, br, edge_index):
    vec2 = vec.reshape(N, D3)
    jj = edge_index[0]
    ii = edge_index[1]
    gx, gv, bpk, bcnt = _gather_call(x, vec2, jj, ii)
    mv0, mv1, mv2, mx = _edge_mlp(gx, gv, edge_rbf,
                                  edge_vector[:, 0:1], edge_vector[:, 1:2],
                                  edge_vector[:, 2:3],
                                  W1, b1.reshape(1, -1), W2, b2.reshape(1, -1),
                                  Wr, br.reshape(1, -1))
    dv0, dv1, dv2c, dx = _scatter_call(mv0, mv1, mv2, mx, bpk, bcnt)
    d_vec = jnp.stack([dv0, dv1, dv2c], axis=1)
    return (dx, d_vec)


# speedup vs baseline: 16.9889x; 1.0567x over previous
import functools
import math

import jax
import jax.numpy as jnp
from jax import lax
from jax.experimental import pallas as pl
from jax.experimental.pallas import tpu as pltpu
from jax.experimental.pallas import tpu_sc as plsc

N = 10000
E = 320000
HC = 128
NRBF = 64
D3 = 3 * HC

NC = 2
NS = 16
NW = NC * NS
EP_T = E // NW
EP_S = E // NS
NHALF = N // NC
ACC_ROWS = 5120
TRASH = NHALF
GCH = 80
SCH = 224
BIN_PAD = ((EP_S + SCH - 1) // SCH) * SCH
ICH = 2000

_PACK_SH = 13

_INV3 = 1.0 / math.sqrt(3.0)
_INVH = 1.0 / math.sqrt(HC)


def _gather_bin_kernel(x_hbm, vec_hbm, j_hbm, i_hbm,
                       gx_hbm, gv_hbm, bpk_hbm, bcnt_hbm):
    c = lax.axis_index("c")
    s = lax.axis_index("s")
    wid = c * NS + s

    def bin_phase(ivm, pb, cbuf):
        def memset(k, _):
            pb[pl.ds(k * 16, 16)] = jnp.full((16,), TRASH, jnp.int32)
            return 0

        lax.fori_loop(0, BIN_PAD // 16, memset, 0)

        lo = c * NHALF

        def chunk(ci, cnt):
            pltpu.sync_copy(i_hbm.at[pl.ds(s * EP_S + ci * ICH, ICH)], ivm)

            def body(k, cnt):
                iv = ivm[pl.ds(k * 16, 16)]
                m = (iv >= lo) & (iv < lo + NHALF)
                dl = iv - lo
                ids = s * EP_S + ci * ICH + k * 16 + lax.iota(jnp.int32, 16)
                w = jnp.bitwise_or(jnp.left_shift(ids, _PACK_SH), dl)
                ps = plsc.cumsum(jnp.where(m, jnp.int32(1), jnp.int32(0)))
                pos = cnt + ps - 1
                plsc.store_scatter(pb, [pos], w, mask=m)
                return cnt + jnp.max(ps)

            return lax.fori_loop(0, ICH // 16, body, cnt)

        cnt = lax.fori_loop(0, EP_S // ICH, chunk, jnp.int32(0))

        pltpu.sync_copy(pb, bpk_hbm.at[wid])
        cbuf[...] = jnp.where(lax.iota(jnp.int32, 16) == 0,
                              jnp.full((16,), cnt, jnp.int32),
                              jnp.zeros((16,), jnp.int32))
        pltpu.sync_copy(cbuf, bcnt_hbm.at[wid])

    pl.run_scoped(bin_phase,
                  pltpu.VMEM((ICH,), jnp.int32),
                  pltpu.VMEM((BIN_PAD,), jnp.int32),
                  pltpu.VMEM((16,), jnp.int32))

    def gather_phase(jvm, gx0, gx1, gv0, gv1, sx0, sx1, sv0, sv1):
        pltpu.sync_copy(j_hbm.at[pl.ds(wid * EP_T, EP_T)], jvm)
        gxs, gvs, sxs, svs = (gx0, gx1), (gv0, gv1), (sx0, sx1), (sv0, sv1)

        def issue(t, b):
            idxs = jvm.at[pl.ds(t * GCH, GCH)]
            pltpu.async_copy(x_hbm.at[idxs], gxs[b], sxs[b])
            pltpu.async_copy(vec_hbm.at[idxs], gvs[b], svs[b])

        def finish(t, b):
            base = wid * EP_T + t * GCH
            idxs = jvm.at[pl.ds(t * GCH, GCH)]
            pltpu.make_async_copy(x_hbm.at[idxs], gxs[b], sxs[b]).wait()
            pltpu.make_async_copy(vec_hbm.at[idxs], gvs[b], svs[b]).wait()
            pltpu.sync_copy(gxs[b], gx_hbm.at[pl.ds(base, GCH)])
            pltpu.sync_copy(gvs[b], gv_hbm.at[pl.ds(base, GCH)])

        nch = EP_T // GCH

        def body(t, _):
            @pl.when(t % 2 == 1)
            def _():
                issue(t, 1)
                finish(t - 1, 0)

            @pl.when(t % 2 == 0)
            def _():
                issue(t, 0)
                finish(t - 1, 1)

            return 0

        issue(0, 0)
        lax.fori_loop(1, nch, body, 0)
        finish(nch - 1, (nch - 1) % 2)

    pl.run_scoped(gather_phase,
                  pltpu.VMEM((EP_T,), jnp.int32),
                  pltpu.VMEM((GCH, HC), jnp.float32),
                  pltpu.VMEM((GCH, HC), jnp.float32),
                  pltpu.VMEM((GCH, D3), jnp.float32),
                  pltpu.VMEM((GCH, D3), jnp.float32),
                  pltpu.SemaphoreType.DMA,
                  pltpu.SemaphoreType.DMA,
                  pltpu.SemaphoreType.DMA,
                  pltpu.SemaphoreType.DMA)


def _edge_mlp_kernel(gx_ref, gv_ref, rbf_ref, ev0_ref, ev1_ref, ev2_ref,
                     W1_ref, b1_ref, W2_ref, b2_ref, Wr_ref, br_ref,
                     mv0_ref, mv1_ref, mv2_ref, mx_ref):
    dn = (((1,), (1,)), ((), ()))
    gx = gx_ref[...]
    h = lax.dot_general(gx, W1_ref[...], dn, preferred_element_type=jnp.float32)
    h = h + b1_ref[...]
    h = (h * jax.nn.sigmoid(h)) * (1.0 / 0.6)
    xh = lax.dot_general(h, W2_ref[...], dn, preferred_element_type=jnp.float32)
    xh = xh + b2_ref[...]
    rh = lax.dot_general(rbf_ref[...], Wr_ref[...], dn,
                         preferred_element_type=jnp.float32)
    rh = rh + br_ref[...]
    t = xh * rh * _INV3
    t1 = t[:, :HC]
    t2 = t[:, HC:2 * HC]
    mx_ref[...] = t[:, 2 * HC:]
    evs = (ev0_ref[...], ev1_ref[...], ev2_ref[...])
    mv_refs = (mv0_ref, mv1_ref, mv2_ref)
    for d in range(3):
        mv_refs[d][...] = (
            t1 * gv_ref[:, d * HC:(d + 1) * HC] + t2 * evs[d]) * _INVH


def _scatter_kernel(m0_hbm, m1_hbm, m2_hbm, m3_hbm, bpk_hbm, bcnt_hbm,
                    o0_hbm, o1_hbm, o2_hbm, o3_hbm, acc):
    c = lax.axis_index("c")
    s = lax.axis_index("s")
    wid = c * NS + s

    def outer(pbv, ic0, ic1, dc0, dc1, cbuf, rows0, rows1, zb, sem0, sem1):
        pltpu.sync_copy(bcnt_hbm.at[wid], cbuf)
        cnt = jnp.max(cbuf[...])
        nch = jnp.maximum((cnt + (SCH - 1)) // SCH, 1)
        pltpu.sync_copy(bpk_hbm.at[wid], pbv)

        for r in range(16):
            for q in range(HC // 16):
                zb[r, pl.ds(q * 16, 16)] = jnp.zeros((16,), jnp.float32)

        rows_per_tile = ACC_ROWS // NS
        ics, dcs, rowss, sems = (ic0, ic1), (dc0, dc1), (rows0, rows1), \
            (sem0, sem1)

        for src_hbm, out_hbm in ((m0_hbm, o0_hbm), (m1_hbm, o1_hbm),
                                 (m2_hbm, o2_hbm), (m3_hbm, o3_hbm)):
            def zbody(t, _):
                pltpu.sync_copy(zb,
                                acc.at[pl.ds(s * rows_per_tile + t * 16, 16)])
                return 0

            lax.fori_loop(0, rows_per_tile // 16, zbody, 0)
            plsc.subcore_barrier()

            def issue(k, b):
                for q in range(SCH // 16):
                    w = pbv[pl.ds(k * SCH + q * 16, 16)]
                    ics[b][pl.ds(q * 16, 16)] = lax.shift_right_logical(
                        w, jnp.full((16,), _PACK_SH, jnp.int32))
                    dcs[b][pl.ds(q * 16, 16)] = jnp.bitwise_and(
                        w, (1 << _PACK_SH) - 1)
                pltpu.async_copy(src_hbm.at[ics[b]], rowss[b], sems[b])

            def finish(b):
                pltpu.make_async_copy(src_hbm.at[ics[b]], rowss[b],
                                      sems[b]).wait()
                pltpu.sync_copy(rowss[b], acc.at[dcs[b]], add=True)

            def body(k, _):
                @pl.when(k % 2 == 1)
                def _():
                    issue(k, 1)
                    finish(0)

                @pl.when(k % 2 == 0)
                def _():
                    issue(k, 0)
                    finish(1)

                return 0

            issue(0, 0)
            lax.fori_loop(1, nch, body, 0)

            @pl.when(nch % 2 == 1)
            def _():
                finish(0)

            @pl.when(nch % 2 == 0)
            def _():
                finish(1)

            plsc.subcore_barrier()

            @pl.when(s < 15)
            def _():
                pltpu.sync_copy(acc.at[pl.ds(s * 312, 312)],
                                out_hbm.at[pl.ds(c * NHALF + s * 312, 312)])

            @pl.when(s == 15)
            def _():
                pltpu.sync_copy(acc.at[pl.ds(4680, 320)],
                                out_hbm.at[pl.ds(c * NHALF + 4680, 320)])

            plsc.subcore_barrier()

    pl.run_scoped(outer,
                  pltpu.VMEM((BIN_PAD,), jnp.int32),
                  pltpu.VMEM((SCH,), jnp.int32),
                  pltpu.VMEM((SCH,), jnp.int32),
                  pltpu.VMEM((SCH,), jnp.int32),
                  pltpu.VMEM((SCH,), jnp.int32),
                  pltpu.VMEM((16,), jnp.int32),
                  pltpu.VMEM((SCH, HC), jnp.float32),
                  pltpu.VMEM((SCH, HC), jnp.float32),
                  pltpu.VMEM((16, HC), jnp.float32),
                  pltpu.SemaphoreType.DMA,
                  pltpu.SemaphoreType.DMA)


_SC_MESH = plsc.VectorSubcoreMesh(core_axis_name="c", subcore_axis_name="s")
_SC_PARAMS = pltpu.CompilerParams(needs_layout_passes=False)

_gather_call = functools.partial(
    pl.kernel, mesh=_SC_MESH, compiler_params=_SC_PARAMS,
    out_type=[
        jax.ShapeDtypeStruct((E, HC), jnp.float32),
        jax.ShapeDtypeStruct((E, D3), jnp.float32),
        jax.ShapeDtypeStruct((NW, BIN_PAD), jnp.int32),
        jax.ShapeDtypeStruct((NW, 16), jnp.int32),
    ])(_gather_bin_kernel)

_scatter_call = functools.partial(
    pl.kernel, mesh=_SC_MESH, compiler_params=_SC_PARAMS,
    out_type=[jax.ShapeDtypeStruct((N, HC), jnp.float32)] * 4,
    scratch_types=[pltpu.VMEM_SHARED((ACC_ROWS, HC), jnp.float32)],
    )(_scatter_kernel)

_EB = 2000


def _edge_mlp(gx, gv, rbf, ev0, ev1, ev2, W1, b1, W2, b2, Wr, br):
    grid = (E // _EB,)
    row = lambda m: (m, 0)
    fixed = lambda m: (0, 0)
    return pl.pallas_call(
        _edge_mlp_kernel,
        grid=grid,
        in_specs=[
            pl.BlockSpec((_EB, HC), row),
            pl.BlockSpec((_EB, D3), row),
            pl.BlockSpec((_EB, NRBF), row),
            pl.BlockSpec((_EB, 1), row),
            pl.BlockSpec((_EB, 1), row),
            pl.BlockSpec((_EB, 1), row),
            pl.BlockSpec((HC // 2, HC), fixed),
            pl.BlockSpec((1, HC // 2), fixed),
            pl.BlockSpec((D3, HC // 2), fixed),
            pl.BlockSpec((1, D3), fixed),
            pl.BlockSpec((D3, NRBF), fixed),
            pl.BlockSpec((1, D3), fixed),
        ],
        out_specs=[pl.BlockSpec((_EB, HC), row)] * 4,
        out_shape=[jax.ShapeDtypeStruct((E, HC), jnp.float32)] * 4,
    )(gx, gv, rbf, ev0, ev1, ev2, W1, b1, W2, b2, Wr, br)


def kernel(x, vec, edge_rbf, edge_vector, W1, b1, W2, b2, Wr, br, edge_index):
    vec2 = vec.reshape(N, D3)
    jj = edge_index[0]
    ii = edge_index[1]
    gx, gv, bpk, bcnt = _gather_call(x, vec2, jj, ii)
    mv0, mv1, mv2, mx = _edge_mlp(gx, gv, edge_rbf,
                                  edge_vector[:, 0:1], edge_vector[:, 1:2],
                                  edge_vector[:, 2:3],
                                  W1, b1.reshape(1, -1), W2, b2.reshape(1, -1),
                                  Wr, br.reshape(1, -1))
    dv0, dv1, dv2c, dx = _scatter_call(mv0, mv1, mv2, mx, bpk, bcnt)
    d_vec = jnp.stack([dv0, dv1, dv2c], axis=1)
    return (dx, d_vec)
